# Initial kernel scaffold; baseline (speedup 1.0000x reference)
import jax
import jax.numpy as jnp
from jax.experimental import pallas as pl


def kernel(x, edge_index, W1, b1, W2, b2):
    raise NotImplementedError("write your pallas kernel here")


# speedup vs baseline: 24.2896x; 24.2896x over previous
import functools

import jax
import jax.numpy as jnp
from jax import lax
from jax.experimental import pallas as pl
from jax.experimental.pallas import tpu as pltpu
from jax.experimental.pallas import tpu_sc as plsc

N_NODES = 10000
N_EDGES = 320000
D_FEAT = 128
HIDDEN = 16
D_OUT = 64

NC = 2
NS = 16
NW = NC * NS
CHUNK = 128
JCH = 79
E_PAD = NW * JCH * CHUNK
N_PAD = 10240
DUMMY = N_NODES
RPW = N_PAD // NS

_mesh = plsc.VectorSubcoreMesh(core_axis_name="c", subcore_axis_name="s")
_sc_params = pltpu.CompilerParams(use_tc_tiling_on_sc=False)


def _make_deg_kernel():
    @functools.partial(
        pl.kernel,
        out_type=jax.ShapeDtypeStruct((NC, N_PAD), jnp.float32),
        mesh=_mesh,
        compiler_params=_sc_params,
        scratch_types=[
            pltpu.VMEM((JCH, CHUNK), jnp.int32),
            pltpu.VMEM((CHUNK,), jnp.float32),
            pltpu.VMEM((RPW,), jnp.float32),
            pltpu.VMEM_SHARED((N_PAD,), jnp.float32),
        ],
    )
    def deg_kernel(dst_hbm, out_hbm, dst_v, ones_v, obuf_v, acc_sh):
        c = lax.axis_index("c")
        s = lax.axis_index("s")
        w = c * NS + s

        pltpu.sync_copy(dst_hbm.at[w], dst_v)

        zero16 = jnp.zeros((16,), jnp.float32)
        ones16 = jnp.full((16,), 1.0, jnp.float32)
        for k in range(CHUNK // 16):
            ones_v[pl.ds(k * 16, 16)] = ones16

        def zrow(i, _):
            obuf_v[pl.ds(i * 16, 16)] = zero16
            return 0

        lax.fori_loop(0, RPW // 16, zrow, 0)
        pltpu.sync_copy(obuf_v, acc_sh.at[pl.ds(s * RPW, RPW)])
        plsc.subcore_barrier()

        def chunk(j, _):
            pltpu.sync_copy(ones_v, acc_sh.at[dst_v.at[j]], add=True)
            return 0

        lax.fori_loop(0, JCH, chunk, 0)

        plsc.subcore_barrier()
        pltpu.sync_copy(acc_sh.at[pl.ds(s * RPW, RPW)], obuf_v)
        pltpu.sync_copy(obuf_v, out_hbm.at[c, pl.ds(s * RPW, RPW)])

    return deg_kernel


def _make_scatter_kernel(d_feat: int):
    @functools.partial(
        pl.kernel,
        out_type=jax.ShapeDtypeStruct((NC, N_PAD, d_feat), jnp.float32),
        mesh=_mesh,
        compiler_params=_sc_params,
        scratch_types=[
            pltpu.VMEM((JCH, CHUNK), jnp.int32),
            pltpu.VMEM((JCH, CHUNK), jnp.int32),
            pltpu.VMEM((CHUNK, d_feat), jnp.float32),
            pltpu.VMEM((RPW, d_feat), jnp.float32),
            pltpu.VMEM_SHARED((N_PAD, d_feat), jnp.float32),
            pltpu.SemaphoreType.DMA,
        ],
    )
    def scat_kernel(g_hbm, src_hbm, dst_hbm, out_hbm,
                    src_v, dst_v, rows_v, buf_v, acc_sh, sem):
        c = lax.axis_index("c")
        s = lax.axis_index("s")
        w = c * NS + s

        pltpu.sync_copy(src_hbm.at[w], src_v)
        pltpu.sync_copy(dst_hbm.at[w], dst_v)

        zero16 = jnp.zeros((16,), jnp.float32)

        def zrow(i, _):
            for k in range(d_feat // 16):
                buf_v[i, pl.ds(k * 16, 16)] = zero16
            return 0

        lax.fori_loop(0, RPW, zrow, 0)
        pltpu.sync_copy(buf_v, acc_sh.at[pl.ds(s * RPW, RPW)])
        plsc.subcore_barrier()

        def chunk(j, _):
            pltpu.async_copy(g_hbm.at[src_v.at[j]], rows_v, sem).wait()
            pltpu.sync_copy(rows_v, acc_sh.at[dst_v.at[j]], add=True)
            return 0

        lax.fori_loop(0, JCH, chunk, 0)

        plsc.subcore_barrier()
        pltpu.sync_copy(acc_sh.at[pl.ds(s * RPW, RPW)], buf_v)
        pltpu.sync_copy(buf_v, out_hbm.at[c, pl.ds(s * RPW, RPW)])

    return scat_kernel


_deg_kernel = _make_deg_kernel()
_scat16 = _make_scatter_kernel(HIDDEN)
_scat64 = _make_scatter_kernel(D_OUT)


def _mm1_body(x_ref, w_ref, dinv_ref, g_ref):
    h = jnp.dot(x_ref[...], w_ref[...], preferred_element_type=jnp.float32)
    g_ref[...] = h * dinv_ref[...]


def _mid_body(accp_ref, g1_ref, dinv_ref, b1_ref, w2_ref, g2_ref):
    a = accp_ref[0] + accp_ref[1] + g1_ref[...]
    z = jnp.maximum(a * dinv_ref[...] + b1_ref[...], 0.0)
    g2_ref[...] = (
        jnp.dot(z, w2_ref[...], preferred_element_type=jnp.float32)
        * dinv_ref[...]
    )


def _out_body(accp_ref, g2_ref, dinv_ref, b2_ref, out_ref):
    out_ref[...] = (
        (accp_ref[0] + accp_ref[1] + g2_ref[...]) * dinv_ref[...]
        + b2_ref[...]
    )


def kernel(x, edge_index, W1, b1, W2, b2):
    src = edge_index[0]
    dst = edge_index[1]
    pad = E_PAD - N_EDGES
    src3 = jnp.concatenate(
        [src, jnp.zeros((pad,), jnp.int32)]).reshape(NW, JCH, CHUNK)
    dst3 = jnp.concatenate(
        [dst, jnp.full((pad,), DUMMY, jnp.int32)]).reshape(NW, JCH, CHUNK)
    x_pad = jnp.pad(x, ((0, N_PAD - N_NODES), (0, 0)))

    deg_p = _deg_kernel(dst3)
    dinv = lax.rsqrt(deg_p[0] + deg_p[1] + 1.0).reshape(N_PAD, 1)

    g1 = pl.pallas_call(
        _mm1_body,
        out_shape=jax.ShapeDtypeStruct((N_PAD, HIDDEN), jnp.float32),
    )(x_pad, W1, dinv)

    acc1p = _scat16(g1, src3, dst3)

    g2 = pl.pallas_call(
        _mid_body,
        out_shape=jax.ShapeDtypeStruct((N_PAD, D_OUT), jnp.float32),
    )(acc1p, g1, dinv, b1.reshape(1, HIDDEN), W2)

    acc2p = _scat64(g2, src3, dst3)

    out = pl.pallas_call(
        _out_body,
        out_shape=jax.ShapeDtypeStruct((N_PAD, D_OUT), jnp.float32),
    )(acc2p, g2, dinv, b2.reshape(1, D_OUT))

    return out[:N_NODES]


# speedup vs baseline: 39.2368x; 1.6154x over previous
import functools

import jax
import jax.numpy as jnp
from jax import lax
from jax.experimental import pallas as pl
from jax.experimental.pallas import tpu as pltpu
from jax.experimental.pallas import tpu_sc as plsc

N_NODES = 10000
N_EDGES = 320000
D_FEAT = 128
HIDDEN = 16
D_OUT = 64

NC = 2
NS = 16
NW = NC * NS
CHUNK = 128
JCH = 80
NBUF = 4
E_PAD = NW * JCH * CHUNK
N_PAD = 10240
DUMMY = N_NODES
RPW = N_PAD // NS

_mesh = plsc.VectorSubcoreMesh(core_axis_name="c", subcore_axis_name="s")
_sc_params = pltpu.CompilerParams(use_tc_tiling_on_sc=False)


def _make_deg_kernel():
    @functools.partial(
        pl.kernel,
        out_type=jax.ShapeDtypeStruct((NC, N_PAD), jnp.float32),
        mesh=_mesh,
        compiler_params=_sc_params,
        scratch_types=[
            pltpu.VMEM((JCH, CHUNK), jnp.int32),
            pltpu.VMEM((CHUNK,), jnp.float32),
            pltpu.VMEM((RPW,), jnp.float32),
            pltpu.VMEM_SHARED((N_PAD,), jnp.float32),
        ],
    )
    def deg_kernel(dst_hbm, out_hbm, dst_v, ones_v, obuf_v, acc_sh):
        c = lax.axis_index("c")
        s = lax.axis_index("s")
        w = c * NS + s

        pltpu.sync_copy(dst_hbm.at[w], dst_v)

        zero16 = jnp.zeros((16,), jnp.float32)
        ones16 = jnp.full((16,), 1.0, jnp.float32)
        for k in range(CHUNK // 16):
            ones_v[pl.ds(k * 16, 16)] = ones16

        def zrow(i, _):
            obuf_v[pl.ds(i * 16, 16)] = zero16
            return 0

        lax.fori_loop(0, RPW // 16, zrow, 0)
        pltpu.sync_copy(obuf_v, acc_sh.at[pl.ds(s * RPW, RPW)])
        plsc.subcore_barrier()

        def chunk(j, _):
            pltpu.sync_copy(ones_v, acc_sh.at[dst_v.at[j]], add=True)
            return 0

        lax.fori_loop(0, JCH, chunk, 0)

        plsc.subcore_barrier()
        pltpu.sync_copy(acc_sh.at[pl.ds(s * RPW, RPW)], obuf_v)
        pltpu.sync_copy(obuf_v, out_hbm.at[c, pl.ds(s * RPW, RPW)])

    return deg_kernel


def _make_scatter_kernel(d_feat: int):
    @functools.partial(
        pl.kernel,
        out_type=jax.ShapeDtypeStruct((NC, N_PAD, d_feat), jnp.float32),
        mesh=_mesh,
        compiler_params=_sc_params,
        scratch_types=[
            pltpu.VMEM((JCH, CHUNK), jnp.int32),
            pltpu.VMEM((JCH, CHUNK), jnp.int32),
            pltpu.VMEM((NBUF, CHUNK, d_feat), jnp.float32),
            pltpu.VMEM((RPW, d_feat), jnp.float32),
            pltpu.VMEM_SHARED((N_PAD, d_feat), jnp.float32),
        ] + [pltpu.SemaphoreType.DMA] * NBUF,
    )
    def scat_kernel(g_hbm, src_hbm, dst_hbm, out_hbm,
                    src_v, dst_v, rows_v, buf_v, acc_sh, *sems):
        c = lax.axis_index("c")
        s = lax.axis_index("s")
        w = c * NS + s

        pltpu.sync_copy(src_hbm.at[w], src_v)
        pltpu.sync_copy(dst_hbm.at[w], dst_v)

        zero16 = jnp.zeros((16,), jnp.float32)

        def zrow(i, _):
            for k in range(d_feat // 16):
                buf_v[i, pl.ds(k * 16, 16)] = zero16
            return 0

        lax.fori_loop(0, RPW, zrow, 0)
        pltpu.sync_copy(buf_v, acc_sh.at[pl.ds(s * RPW, RPW)])
        plsc.subcore_barrier()

        for b in range(NBUF):
            pltpu.async_copy(g_hbm.at[src_v.at[b]], rows_v.at[b], sems[b])

        def group(g, _):
            for b in range(NBUF):
                j = g * NBUF + b
                pltpu.make_async_copy(
                    g_hbm.at[src_v.at[j]], rows_v.at[b], sems[b]).wait()
                pltpu.sync_copy(rows_v.at[b], acc_sh.at[dst_v.at[j]], add=True)
                pltpu.async_copy(
                    g_hbm.at[src_v.at[j + NBUF]], rows_v.at[b], sems[b])
            return 0

        lax.fori_loop(0, JCH // NBUF - 1, group, 0)
        for b in range(NBUF):
            j = JCH - NBUF + b
            pltpu.make_async_copy(
                g_hbm.at[src_v.at[j]], rows_v.at[b], sems[b]).wait()
            pltpu.sync_copy(rows_v.at[b], acc_sh.at[dst_v.at[j]], add=True)

        plsc.subcore_barrier()
        pltpu.sync_copy(acc_sh.at[pl.ds(s * RPW, RPW)], buf_v)
        pltpu.sync_copy(buf_v, out_hbm.at[c, pl.ds(s * RPW, RPW)])

    return scat_kernel


_deg_kernel = _make_deg_kernel()
_scat16 = _make_scatter_kernel(HIDDEN)


def _mm1_body(x_ref, w_ref, dinv_ref, g_ref):
    h = jnp.dot(x_ref[...], w_ref[...], preferred_element_type=jnp.float32)
    g_ref[...] = h * dinv_ref[...]


def _mid_body(accp_ref, g1_ref, dinv_ref, b1_ref, gz_ref):
    a = accp_ref[0] + accp_ref[1] + g1_ref[...]
    z = jnp.maximum(a * dinv_ref[...] + b1_ref[...], 0.0)
    gz_ref[...] = z * dinv_ref[...]


def _out_body(accp_ref, gz_ref, dinv_ref, b2_ref, w2_ref, out_ref):
    a = accp_ref[0] + accp_ref[1] + gz_ref[...]
    out_ref[...] = (
        jnp.dot(a, w2_ref[...], preferred_element_type=jnp.float32)
        * dinv_ref[...]
        + b2_ref[...]
    )


def kernel(x, edge_index, W1, b1, W2, b2):
    src = edge_index[0]
    dst = edge_index[1]
    pad = E_PAD - N_EDGES
    src3 = jnp.concatenate(
        [src, jnp.zeros((pad,), jnp.int32)]).reshape(NW, JCH, CHUNK)
    dst3 = jnp.concatenate(
        [dst, jnp.full((pad,), DUMMY, jnp.int32)]).reshape(NW, JCH, CHUNK)
    x_pad = jnp.pad(x, ((0, N_PAD - N_NODES), (0, 0)))

    deg_p = _deg_kernel(dst3)
    dinv = lax.rsqrt(deg_p[0] + deg_p[1] + 1.0).reshape(N_PAD, 1)

    g1 = pl.pallas_call(
        _mm1_body,
        out_shape=jax.ShapeDtypeStruct((N_PAD, HIDDEN), jnp.float32),
    )(x_pad, W1, dinv)

    acc1p = _scat16(g1, src3, dst3)

    gz = pl.pallas_call(
        _mid_body,
        out_shape=jax.ShapeDtypeStruct((N_PAD, HIDDEN), jnp.float32),
    )(acc1p, g1, dinv, b1.reshape(1, HIDDEN))

    acczp = _scat16(gz, src3, dst3)

    out = pl.pallas_call(
        _out_body,
        out_shape=jax.ShapeDtypeStruct((N_PAD, D_OUT), jnp.float32),
    )(acczp, gz, dinv, b2.reshape(1, D_OUT), W2)

    return out[:N_NODES]


# speedup vs baseline: 39.7928x; 1.0142x over previous
import functools

import jax
import jax.numpy as jnp
from jax import lax
from jax.experimental import pallas as pl
from jax.experimental.pallas import tpu as pltpu
from jax.experimental.pallas import tpu_sc as plsc

N_NODES = 10000
N_EDGES = 320000
D_FEAT = 128
HIDDEN = 16
D_OUT = 64

NC = 2
NS = 16
CHUNK = 128
K = 4
NCHUNK = 2632
E_PAD = NCHUNK * CHUNK
N_PAD = 10240
DUMMY = N_NODES
RPW = N_PAD // NS

J0, J1 = 104, 56
G0, G1 = J0 // K, J1 // K
JD0, JD1 = 90, 73

_mesh = plsc.VectorSubcoreMesh(core_axis_name="c", subcore_axis_name="s")
_sc_params = pltpu.CompilerParams(use_tc_tiling_on_sc=False)


def _make_deg_kernel():
    @functools.partial(
        pl.kernel,
        out_type=jax.ShapeDtypeStruct((NC, N_PAD), jnp.float32),
        mesh=_mesh,
        compiler_params=_sc_params,
        scratch_types=[
            pltpu.VMEM((JD0, CHUNK), jnp.int32),
            pltpu.VMEM((CHUNK,), jnp.float32),
            pltpu.VMEM((RPW,), jnp.float32),
            pltpu.VMEM_SHARED((N_PAD,), jnp.float32),
            pltpu.SemaphoreType.DMA,
        ],
    )
    def deg_kernel(dst_hbm, out_hbm, dst_v, ones_v, obuf_v, acc_sh, sem):
        c = lax.axis_index("c")
        s = lax.axis_index("s")
        j_n = jnp.where(c == 0, JD0, JD1)
        base = jnp.where(c == 0, s * JD0, NS * JD0 + s * JD1)

        pltpu.sync_copy(dst_hbm.at[pl.ds(base, JD0)], dst_v)

        zero16 = jnp.zeros((16,), jnp.float32)
        ones16 = jnp.full((16,), 1.0, jnp.float32)
        for k in range(CHUNK // 16):
            ones_v[pl.ds(k * 16, 16)] = ones16

        def zrow(i, _):
            obuf_v[pl.ds(i * 16, 16)] = zero16
            return 0

        lax.fori_loop(0, RPW // 16, zrow, 0)
        pltpu.sync_copy(obuf_v, acc_sh.at[pl.ds(s * RPW, RPW)])
        plsc.subcore_barrier()

        def group(g, _):
            for b in range(8):
                pltpu.async_copy(ones_v, acc_sh.at[dst_v.at[g * 8 + b]],
                                 sem, add=True)
            for b in range(8):
                pltpu.make_async_copy(ones_v, acc_sh.at[dst_v.at[0]],
                                      sem).wait()
            return 0

        lax.fori_loop(0, j_n // 8, group, 0)

        def tail(j, _):
            pltpu.sync_copy(ones_v, acc_sh.at[dst_v.at[j]], add=True)
            return 0

        lax.fori_loop((j_n // 8) * 8, j_n, tail, 0)

        plsc.subcore_barrier()
        pltpu.sync_copy(acc_sh.at[pl.ds(s * RPW, RPW)], obuf_v)
        pltpu.sync_copy(obuf_v, out_hbm.at[c, pl.ds(s * RPW, RPW)])

    return deg_kernel


def _make_scatter_kernel(d_feat: int):
    @functools.partial(
        pl.kernel,
        out_type=jax.ShapeDtypeStruct((NC, N_PAD, d_feat), jnp.float32),
        mesh=_mesh,
        compiler_params=_sc_params,
        scratch_types=[
            pltpu.VMEM((J0, CHUNK), jnp.int32),
            pltpu.VMEM((J0, CHUNK), jnp.int32),
            pltpu.VMEM((2, K, CHUNK, d_feat), jnp.float32),
            pltpu.VMEM((RPW, d_feat), jnp.float32),
            pltpu.VMEM_SHARED((N_PAD, d_feat), jnp.float32),
            pltpu.SemaphoreType.DMA,
            pltpu.SemaphoreType.DMA,
        ],
    )
    def scat_kernel(g_hbm, src_hbm, dst_hbm, out_hbm,
                    src_v, dst_v, rows_v, buf_v, acc_sh, gsem, ssem):
        c = lax.axis_index("c")
        s = lax.axis_index("s")
        g_n = jnp.where(c == 0, G0, G1)
        base = jnp.where(c == 0, s * J0, NS * J0 + s * J1)

        pltpu.sync_copy(src_hbm.at[pl.ds(base, J0)], src_v)
        pltpu.sync_copy(dst_hbm.at[pl.ds(base, J0)], dst_v)

        zero16 = jnp.zeros((16,), jnp.float32)

        def zrow(i, _):
            for k in range(d_feat // 16):
                buf_v[i, pl.ds(k * 16, 16)] = zero16
            return 0

        lax.fori_loop(0, RPW, zrow, 0)
        pltpu.sync_copy(buf_v, acc_sh.at[pl.ds(s * RPW, RPW)])
        plsc.subcore_barrier()

        for b in range(K):
            pltpu.async_copy(g_hbm.at[src_v.at[b]], rows_v.at[0, b], gsem)

        def pair(g2, _):
            for p in range(2):
                g = g2 * 2 + p
                for b in range(K):
                    pltpu.make_async_copy(
                        g_hbm.at[src_v.at[0]], rows_v.at[p, b], gsem).wait()
                for b in range(K):
                    pltpu.async_copy(rows_v.at[p, b],
                                     acc_sh.at[dst_v.at[g * K + b]],
                                     ssem, add=True)
                q = 1 - p

                @pl.when(g >= 1)
                def _drain():
                    for b in range(K):
                        pltpu.make_async_copy(
                            g_hbm.at[src_v.at[0]], rows_v.at[q, b],
                            ssem).wait()

                @pl.when(g + 1 < g_n)
                def _refill():
                    for b in range(K):
                        pltpu.async_copy(
                            g_hbm.at[src_v.at[(g + 1) * K + b]],
                            rows_v.at[q, b], gsem)
            return 0

        lax.fori_loop(0, g_n // 2, pair, 0)
        for b in range(K):
            pltpu.make_async_copy(
                g_hbm.at[src_v.at[0]], rows_v.at[0, b], ssem).wait()

        plsc.subcore_barrier()
        pltpu.sync_copy(acc_sh.at[pl.ds(s * RPW, RPW)], buf_v)
        pltpu.sync_copy(buf_v, out_hbm.at[c, pl.ds(s * RPW, RPW)])

    return scat_kernel


_deg_kernel = _make_deg_kernel()
_scat16 = _make_scatter_kernel(HIDDEN)


def _mm1_body(x_ref, w_ref, dinv_ref, g_ref):
    h = jnp.dot(x_ref[...], w_ref[...], preferred_element_type=jnp.float32)
    g_ref[...] = h * dinv_ref[...]


def _mid_body(accp_ref, g1_ref, dinv_ref, b1_ref, gz_ref):
    a = (accp_ref[0] + accp_ref[1])[:N_NODES] + g1_ref[...]
    z = jnp.maximum(a * dinv_ref[...] + b1_ref[...], 0.0)
    gz_ref[...] = z * dinv_ref[...]


def _out_body(accp_ref, gz_ref, dinv_ref, b2_ref, w2_ref, out_ref):
    a = (accp_ref[0] + accp_ref[1])[:N_NODES] + gz_ref[...]
    out_ref[...] = (
        jnp.dot(a, w2_ref[...], preferred_element_type=jnp.float32)
        * dinv_ref[...]
        + b2_ref[...]
    )


def kernel(x, edge_index, W1, b1, W2, b2):
    src = edge_index[0]
    dst = edge_index[1]
    pad = E_PAD - N_EDGES
    src2 = jnp.concatenate(
        [src, jnp.zeros((pad,), jnp.int32)]).reshape(NCHUNK, CHUNK)
    dst2 = jnp.concatenate(
        [dst, jnp.full((pad,), DUMMY, jnp.int32)]).reshape(NCHUNK, CHUNK)

    deg_p = _deg_kernel(dst2)
    dinv = lax.rsqrt(
        deg_p[0, :N_NODES] + deg_p[1, :N_NODES] + 1.0).reshape(N_NODES, 1)

    g1 = pl.pallas_call(
        _mm1_body,
        out_shape=jax.ShapeDtypeStruct((N_NODES, HIDDEN), jnp.float32),
    )(x, W1, dinv)

    acc1p = _scat16(g1, src2, dst2)

    gz = pl.pallas_call(
        _mid_body,
        out_shape=jax.ShapeDtypeStruct((N_NODES, HIDDEN), jnp.float32),
    )(acc1p, g1, dinv, b1.reshape(1, HIDDEN))

    acczp = _scat16(gz, src2, dst2)

    out = pl.pallas_call(
        _out_body,
        out_shape=jax.ShapeDtypeStruct((N_NODES, D_OUT), jnp.float32),
    )(acczp, gz, dinv, b2.reshape(1, D_OUT), W2)

    return out


# speedup vs baseline: 40.0745x; 1.0071x over previous
import functools

import jax
import jax.numpy as jnp
from jax import lax
from jax.experimental import pallas as pl
from jax.experimental.pallas import tpu as pltpu
from jax.experimental.pallas import tpu_sc as plsc

N_NODES = 10000
N_EDGES = 320000
D_FEAT = 128
HIDDEN = 16
D_OUT = 64

NC = 2
NS = 16
CHUNK = 128
K = 4
NCHUNK = 2696
E_PAD = NCHUNK * CHUNK
N_PAD = 10240
DUMMY = N_NODES
RPW = N_PAD // NS

J0, J1 = 120, 40
G0, G1 = J0 // K, J1 // K
JD0, JD1 = 124, 39

_mesh = plsc.VectorSubcoreMesh(core_axis_name="c", subcore_axis_name="s")
_sc_params = pltpu.CompilerParams(use_tc_tiling_on_sc=False)


def _make_deg_kernel():
    @functools.partial(
        pl.kernel,
        out_type=jax.ShapeDtypeStruct((NC, N_PAD), jnp.float32),
        mesh=_mesh,
        compiler_params=_sc_params,
        scratch_types=[
            pltpu.VMEM((JD0, CHUNK), jnp.int32),
            pltpu.VMEM((CHUNK,), jnp.float32),
            pltpu.VMEM((RPW,), jnp.float32),
            pltpu.VMEM_SHARED((N_PAD,), jnp.float32),
            pltpu.SemaphoreType.DMA,
        ],
    )
    def deg_kernel(dst_hbm, out_hbm, dst_v, ones_v, obuf_v, acc_sh, sem):
        c = lax.axis_index("c")
        s = lax.axis_index("s")
        j_n = jnp.where(c == 0, JD0, JD1)
        base = jnp.where(c == 0, s * JD0, NS * JD0 + s * JD1)

        pltpu.sync_copy(dst_hbm.at[pl.ds(base, JD0)], dst_v)

        zero16 = jnp.zeros((16,), jnp.float32)
        ones16 = jnp.full((16,), 1.0, jnp.float32)
        for k in range(CHUNK // 16):
            ones_v[pl.ds(k * 16, 16)] = ones16

        def zrow(i, _):
            obuf_v[pl.ds(i * 16, 16)] = zero16
            return 0

        lax.fori_loop(0, RPW // 16, zrow, 0)
        pltpu.sync_copy(obuf_v, acc_sh.at[pl.ds(s * RPW, RPW)])
        plsc.subcore_barrier()

        def group(g, _):
            for b in range(8):
                pltpu.async_copy(ones_v, acc_sh.at[dst_v.at[g * 8 + b]],
                                 sem, add=True)
            for b in range(8):
                pltpu.make_async_copy(ones_v, acc_sh.at[dst_v.at[0]],
                                      sem).wait()
            return 0

        lax.fori_loop(0, j_n // 8, group, 0)

        def tail(j, _):
            pltpu.sync_copy(ones_v, acc_sh.at[dst_v.at[j]], add=True)
            return 0

        lax.fori_loop((j_n // 8) * 8, j_n, tail, 0)

        plsc.subcore_barrier()
        pltpu.sync_copy(acc_sh.at[pl.ds(s * RPW, RPW)], obuf_v)
        pltpu.sync_copy(obuf_v, out_hbm.at[c, pl.ds(s * RPW, RPW)])

    return deg_kernel


def _make_scatter_kernel(d_feat: int):
    @functools.partial(
        pl.kernel,
        out_type=jax.ShapeDtypeStruct((NC, N_PAD, d_feat), jnp.float32),
        mesh=_mesh,
        compiler_params=_sc_params,
        scratch_types=[
            pltpu.VMEM((J0, CHUNK), jnp.int32),
            pltpu.VMEM((J0, CHUNK), jnp.int32),
            pltpu.VMEM((2, K, CHUNK, d_feat), jnp.float32),
            pltpu.VMEM((RPW, d_feat), jnp.float32),
            pltpu.VMEM_SHARED((N_PAD, d_feat), jnp.float32),
            pltpu.SemaphoreType.DMA,
            pltpu.SemaphoreType.DMA,
        ],
    )
    def scat_kernel(g_hbm, src_hbm, dst_hbm, out_hbm,
                    src_v, dst_v, rows_v, buf_v, acc_sh, gsem, ssem):
        c = lax.axis_index("c")
        s = lax.axis_index("s")
        g_n = jnp.where(c == 0, G0, G1)
        base = jnp.where(c == 0, s * J0, NS * J0 + s * J1)

        pltpu.sync_copy(src_hbm.at[pl.ds(base, J0)], src_v)
        pltpu.sync_copy(dst_hbm.at[pl.ds(base, J0)], dst_v)

        zero16 = jnp.zeros((16,), jnp.float32)

        def zrow(i, _):
            for k in range(d_feat // 16):
                buf_v[i, pl.ds(k * 16, 16)] = zero16
            return 0

        lax.fori_loop(0, RPW, zrow, 0)
        pltpu.sync_copy(buf_v, acc_sh.at[pl.ds(s * RPW, RPW)])
        plsc.subcore_barrier()

        for b in range(K):
            pltpu.async_copy(g_hbm.at[src_v.at[b]], rows_v.at[0, b], gsem)

        def pair(g2, _):
            for p in range(2):
                g = g2 * 2 + p
                for b in range(K):
                    pltpu.make_async_copy(
                        g_hbm.at[src_v.at[0]], rows_v.at[p, b], gsem).wait()
                for b in range(K):
                    pltpu.async_copy(rows_v.at[p, b],
                                     acc_sh.at[dst_v.at[g * K + b]],
                                     ssem, add=True)
                q = 1 - p

                @pl.when(g >= 1)
                def _drain():
                    for b in range(K):
                        pltpu.make_async_copy(
                            g_hbm.at[src_v.at[0]], rows_v.at[q, b],
                            ssem).wait()

                @pl.when(g + 1 < g_n)
                def _refill():
                    for b in range(K):
                        pltpu.async_copy(
                            g_hbm.at[src_v.at[(g + 1) * K + b]],
                            rows_v.at[q, b], gsem)
            return 0

        lax.fori_loop(0, g_n // 2, pair, 0)
        for b in range(K):
            pltpu.make_async_copy(
                g_hbm.at[src_v.at[0]], rows_v.at[0, b], ssem).wait()

        plsc.subcore_barrier()
        pltpu.sync_copy(acc_sh.at[pl.ds(s * RPW, RPW)], buf_v)
        pltpu.sync_copy(buf_v, out_hbm.at[c, pl.ds(s * RPW, RPW)])

    return scat_kernel


_deg_kernel = _make_deg_kernel()
_scat16 = _make_scatter_kernel(HIDDEN)


def _mm1_body(x_ref, w_ref, h_ref):
    h_ref[...] = jnp.dot(x_ref[...], w_ref[...],
                         preferred_element_type=jnp.float32)


def _dinv_g1_body(degp_ref, h_ref, dinv_ref, g_ref):
    deg = degp_ref[0, :N_NODES] + degp_ref[1, :N_NODES] + 1.0
    dinv = lax.rsqrt(deg).reshape(N_NODES, 1)
    dinv_ref[...] = dinv
    g_ref[...] = h_ref[...] * dinv


def _mid_body(accp_ref, g1_ref, dinv_ref, b1_ref, gz_ref):
    a = (accp_ref[0] + accp_ref[1])[:N_NODES] + g1_ref[...]
    z = jnp.maximum(a * dinv_ref[...] + b1_ref[...], 0.0)
    gz_ref[...] = z * dinv_ref[...]


def _out_body(accp_ref, gz_ref, dinv_ref, b2_ref, w2_ref, out_ref):
    a = (accp_ref[0] + accp_ref[1])[:N_NODES] + gz_ref[...]
    out_ref[...] = (
        jnp.dot(a, w2_ref[...], preferred_element_type=jnp.float32)
        * dinv_ref[...]
        + b2_ref[...]
    )


def kernel(x, edge_index, W1, b1, W2, b2):
    src = edge_index[0]
    dst = edge_index[1]
    pad = E_PAD - N_EDGES
    src2 = jnp.concatenate(
        [src, jnp.zeros((pad,), jnp.int32)]).reshape(NCHUNK, CHUNK)
    dst2 = jnp.concatenate(
        [dst, jnp.full((pad,), DUMMY, jnp.int32)]).reshape(NCHUNK, CHUNK)

    deg_p = _deg_kernel(dst2)
    h1 = pl.pallas_call(
        _mm1_body,
        out_shape=jax.ShapeDtypeStruct((N_NODES, HIDDEN), jnp.float32),
    )(x, W1)

    dinv, g1 = pl.pallas_call(
        _dinv_g1_body,
        out_shape=[
            jax.ShapeDtypeStruct((N_NODES, 1), jnp.float32),
            jax.ShapeDtypeStruct((N_NODES, HIDDEN), jnp.float32),
        ],
    )(deg_p, h1)

    acc1p = _scat16(g1, src2, dst2)

    gz = pl.pallas_call(
        _mid_body,
        out_shape=jax.ShapeDtypeStruct((N_NODES, HIDDEN), jnp.float32),
    )(acc1p, g1, dinv, b1.reshape(1, HIDDEN))

    acczp = _scat16(gz, src2, dst2)

    out = pl.pallas_call(
        _out_body,
        out_shape=jax.ShapeDtypeStruct((N_NODES, D_OUT), jnp.float32),
    )(acczp, gz, dinv, b2.reshape(1, D_OUT), W2)

    return out


# speedup vs baseline: 44.3199x; 1.1059x over previous
import functools

import jax
import jax.numpy as jnp
from jax import lax
from jax.experimental import pallas as pl
from jax.experimental.pallas import tpu as pltpu
from jax.experimental.pallas import tpu_sc as plsc

N_NODES = 10000
N_EDGES = 320000
D_FEAT = 128
HIDDEN = 16
D_OUT = 64

NC = 2
NS = 16
CHUNK = 128
K = 4
NCHUNK = 2696
E_PAD = NCHUNK * CHUNK
N_PAD = 10240
DUMMY = N_NODES
RPW = N_PAD // NS

J0, J1 = 120, 40
G0, G1 = J0 // K, J1 // K
SPL0, SPL1 = 12, 4
GROWS = N_NODES // NS
JD0, JD1 = 124, 39

_mesh = plsc.VectorSubcoreMesh(core_axis_name="c", subcore_axis_name="s")
_sc_params = pltpu.CompilerParams(use_tc_tiling_on_sc=False)


def _make_deg_kernel():
    @functools.partial(
        pl.kernel,
        out_type=jax.ShapeDtypeStruct((NC, N_PAD), jnp.float32),
        mesh=_mesh,
        compiler_params=_sc_params,
        scratch_types=[
            pltpu.VMEM((JD0, CHUNK), jnp.int32),
            pltpu.VMEM((CHUNK,), jnp.float32),
            pltpu.VMEM((RPW,), jnp.float32),
            pltpu.VMEM_SHARED((N_PAD,), jnp.float32),
            pltpu.SemaphoreType.DMA,
        ],
    )
    def deg_kernel(dst_hbm, out_hbm, dst_v, ones_v, obuf_v, acc_sh, sem):
        c = lax.axis_index("c")
        s = lax.axis_index("s")
        j_n = jnp.where(c == 0, JD0, JD1)
        base = jnp.where(c == 0, s * JD0, NS * JD0 + s * JD1)

        pltpu.sync_copy(dst_hbm.at[pl.ds(base, JD0)], dst_v)

        zero16 = jnp.zeros((16,), jnp.float32)
        ones16 = jnp.full((16,), 1.0, jnp.float32)
        for k in range(CHUNK // 16):
            ones_v[pl.ds(k * 16, 16)] = ones16

        def zrow(i, _):
            obuf_v[pl.ds(i * 16, 16)] = zero16
            return 0

        lax.fori_loop(0, RPW // 16, zrow, 0)
        pltpu.sync_copy(obuf_v, acc_sh.at[pl.ds(s * RPW, RPW)])
        plsc.subcore_barrier()

        def group(g, _):
            for b in range(8):
                pltpu.async_copy(ones_v, acc_sh.at[dst_v.at[g * 8 + b]],
                                 sem, add=True)
            for b in range(8):
                pltpu.make_async_copy(ones_v, acc_sh.at[dst_v.at[0]],
                                      sem).wait()
            return 0

        lax.fori_loop(0, j_n // 8, group, 0)

        def tail(j, _):
            pltpu.sync_copy(ones_v, acc_sh.at[dst_v.at[j]], add=True)
            return 0

        lax.fori_loop((j_n // 8) * 8, j_n, tail, 0)

        plsc.subcore_barrier()
        pltpu.sync_copy(acc_sh.at[pl.ds(s * RPW, RPW)], obuf_v)
        pltpu.sync_copy(obuf_v, out_hbm.at[c, pl.ds(s * RPW, RPW)])

    return deg_kernel


def _make_scatter_kernel(d_feat: int):
    @functools.partial(
        pl.kernel,
        out_type=jax.ShapeDtypeStruct((NC, N_PAD, d_feat), jnp.float32),
        mesh=_mesh,
        compiler_params=_sc_params,
        scratch_types=[
            pltpu.VMEM((J0, CHUNK), jnp.int32),
            pltpu.VMEM((J0, CHUNK), jnp.int32),
            pltpu.VMEM((2, K, CHUNK, d_feat), jnp.float32),
            pltpu.VMEM((RPW, d_feat), jnp.float32),
            pltpu.VMEM_SHARED((N_PAD, d_feat), jnp.float32),
            pltpu.VMEM_SHARED((N_NODES, d_feat), jnp.float32),
            pltpu.SemaphoreType.DMA,
            pltpu.SemaphoreType.DMA,
        ],
    )
    def scat_kernel(g_hbm, src_hbm, dst_hbm, out_hbm,
                    src_v, dst_v, rows_v, buf_v, acc_sh, g_sh, gsem, ssem):
        c = lax.axis_index("c")
        s = lax.axis_index("s")
        g_n = jnp.where(c == 0, G0, G1)
        spl = jnp.where(c == 0, SPL0, SPL1)
        base = jnp.where(c == 0, s * J0, NS * J0 + s * J1)

        pltpu.sync_copy(src_hbm.at[pl.ds(base, J0)], src_v)
        pltpu.sync_copy(dst_hbm.at[pl.ds(base, J0)], dst_v)

        pltpu.sync_copy(g_hbm.at[pl.ds(s * GROWS, GROWS)],
                        g_sh.at[pl.ds(s * GROWS, GROWS)])

        zero16 = jnp.zeros((16,), jnp.float32)

        def zrow(i, _):
            for k in range(d_feat // 16):
                buf_v[i, pl.ds(k * 16, 16)] = zero16
            return 0

        lax.fori_loop(0, RPW, zrow, 0)
        pltpu.sync_copy(buf_v, acc_sh.at[pl.ds(s * RPW, RPW)])
        plsc.subcore_barrier()

        @pl.when(spl > 0)
        def _pro_sh():
            for b in range(K):
                pltpu.async_copy(g_sh.at[src_v.at[b]], rows_v.at[0, b], gsem)

        @pl.when(spl <= 0)
        def _pro_hbm():
            for b in range(K):
                pltpu.async_copy(g_hbm.at[src_v.at[b]], rows_v.at[0, b], gsem)

        def pair(g2, _):
            for p in range(2):
                g = g2 * 2 + p
                for b in range(K):
                    pltpu.make_async_copy(
                        g_hbm.at[src_v.at[0]], rows_v.at[p, b], gsem).wait()
                for b in range(K):
                    pltpu.async_copy(rows_v.at[p, b],
                                     acc_sh.at[dst_v.at[g * K + b]],
                                     ssem, add=True)
                q = 1 - p

                @pl.when(g >= 1)
                def _drain():
                    for b in range(K):
                        pltpu.make_async_copy(
                            g_hbm.at[src_v.at[0]], rows_v.at[q, b],
                            ssem).wait()

                @pl.when((g + 1 < g_n) & (g + 1 < spl))
                def _refill_sh():
                    for b in range(K):
                        pltpu.async_copy(
                            g_sh.at[src_v.at[(g + 1) * K + b]],
                            rows_v.at[q, b], gsem)

                @pl.when((g + 1 < g_n) & (g + 1 >= spl))
                def _refill_hbm():
                    for b in range(K):
                        pltpu.async_copy(
                            g_hbm.at[src_v.at[(g + 1) * K + b]],
                            rows_v.at[q, b], gsem)
            return 0

        lax.fori_loop(0, g_n // 2, pair, 0)
        for b in range(K):
            pltpu.make_async_copy(
                g_hbm.at[src_v.at[0]], rows_v.at[0, b], ssem).wait()

        plsc.subcore_barrier()
        pltpu.sync_copy(acc_sh.at[pl.ds(s * RPW, RPW)], buf_v)
        pltpu.sync_copy(buf_v, out_hbm.at[c, pl.ds(s * RPW, RPW)])

    return scat_kernel


_deg_kernel = _make_deg_kernel()
_scat16 = _make_scatter_kernel(HIDDEN)


def _mm1_body(x_ref, w_ref, h_ref):
    h_ref[...] = jnp.dot(x_ref[...], w_ref[...],
                         preferred_element_type=jnp.float32)


def _dinv_g1_body(degp_ref, h_ref, dinv_ref, g_ref):
    deg = degp_ref[0, :N_NODES] + degp_ref[1, :N_NODES] + 1.0
    dinv = lax.rsqrt(deg).reshape(N_NODES, 1)
    dinv_ref[...] = dinv
    g_ref[...] = h_ref[...] * dinv


def _mid_body(accp_ref, g1_ref, dinv_ref, b1_ref, gz_ref):
    a = (accp_ref[0] + accp_ref[1])[:N_NODES] + g1_ref[...]
    z = jnp.maximum(a * dinv_ref[...] + b1_ref[...], 0.0)
    gz_ref[...] = z * dinv_ref[...]


def _out_body(accp_ref, gz_ref, dinv_ref, b2_ref, w2_ref, out_ref):
    a = (accp_ref[0] + accp_ref[1])[:N_NODES] + gz_ref[...]
    out_ref[...] = (
        jnp.dot(a, w2_ref[...], preferred_element_type=jnp.float32)
        * dinv_ref[...]
        + b2_ref[...]
    )


def kernel(x, edge_index, W1, b1, W2, b2):
    src = edge_index[0]
    dst = edge_index[1]
    pad = E_PAD - N_EDGES
    src2 = jnp.concatenate(
        [src, jnp.zeros((pad,), jnp.int32)]).reshape(NCHUNK, CHUNK)
    dst2 = jnp.concatenate(
        [dst, jnp.full((pad,), DUMMY, jnp.int32)]).reshape(NCHUNK, CHUNK)

    deg_p = _deg_kernel(dst2)
    h1 = pl.pallas_call(
        _mm1_body,
        out_shape=jax.ShapeDtypeStruct((N_NODES, HIDDEN), jnp.float32),
    )(x, W1)

    dinv, g1 = pl.pallas_call(
        _dinv_g1_body,
        out_shape=[
            jax.ShapeDtypeStruct((N_NODES, 1), jnp.float32),
            jax.ShapeDtypeStruct((N_NODES, HIDDEN), jnp.float32),
        ],
    )(deg_p, h1)

    acc1p = _scat16(g1, src2, dst2)

    gz = pl.pallas_call(
        _mid_body,
        out_shape=jax.ShapeDtypeStruct((N_NODES, HIDDEN), jnp.float32),
    )(acc1p, g1, dinv, b1.reshape(1, HIDDEN))

    acczp = _scat16(gz, src2, dst2)

    out = pl.pallas_call(
        _out_body,
        out_shape=jax.ShapeDtypeStruct((N_NODES, D_OUT), jnp.float32),
    )(acczp, gz, dinv, b2.reshape(1, D_OUT), W2)

    return out


# speedup vs baseline: 45.8521x; 1.0346x over previous
import functools

import jax
import jax.numpy as jnp
from jax import lax
from jax.experimental import pallas as pl
from jax.experimental.pallas import tpu as pltpu
from jax.experimental.pallas import tpu_sc as plsc

N_NODES = 10000
N_EDGES = 320000
D_FEAT = 128
HIDDEN = 16
D_OUT = 64

NC = 2
NS = 16
CHUNK = 128
K = 4
NCHUNK = 2696
E_PAD = NCHUNK * CHUNK
N_PAD = 10240
DUMMY = N_NODES
RPW = N_PAD // NS

J0, J1 = 120, 40
G0, G1 = J0 // K, J1 // K
SPL0, SPL1 = 16, 5
GROWS = N_NODES // NS
JD0, JD1 = 124, 39

_mesh = plsc.VectorSubcoreMesh(core_axis_name="c", subcore_axis_name="s")
_sc_params = pltpu.CompilerParams(use_tc_tiling_on_sc=False)


def _make_deg_kernel():
    @functools.partial(
        pl.kernel,
        out_type=jax.ShapeDtypeStruct((NC, N_PAD), jnp.float32),
        mesh=_mesh,
        compiler_params=_sc_params,
        scratch_types=[
            pltpu.VMEM((JD0, CHUNK), jnp.int32),
            pltpu.VMEM((CHUNK,), jnp.float32),
            pltpu.VMEM((RPW,), jnp.float32),
            pltpu.VMEM_SHARED((N_PAD,), jnp.float32),
            pltpu.SemaphoreType.DMA,
        ],
    )
    def deg_kernel(dst_hbm, out_hbm, dst_v, ones_v, obuf_v, acc_sh, sem):
        c = lax.axis_index("c")
        s = lax.axis_index("s")
        j_n = jnp.where(c == 0, JD0, JD1)
        base = jnp.where(c == 0, s * JD0, NS * JD0 + s * JD1)

        pltpu.sync_copy(dst_hbm.at[pl.ds(base, JD0)], dst_v)

        zero16 = jnp.zeros((16,), jnp.float32)
        ones16 = jnp.full((16,), 1.0, jnp.float32)
        for k in range(CHUNK // 16):
            ones_v[pl.ds(k * 16, 16)] = ones16

        def zrow(i, _):
            obuf_v[pl.ds(i * 16, 16)] = zero16
            return 0

        lax.fori_loop(0, RPW // 16, zrow, 0)
        pltpu.sync_copy(obuf_v, acc_sh.at[pl.ds(s * RPW, RPW)])
        plsc.subcore_barrier()

        def group(g, _):
            for b in range(8):
                pltpu.async_copy(ones_v, acc_sh.at[dst_v.at[g * 8 + b]],
                                 sem, add=True)
            for b in range(8):
                pltpu.make_async_copy(ones_v, acc_sh.at[dst_v.at[0]],
                                      sem).wait()
            return 0

        lax.fori_loop(0, j_n // 8, group, 0)

        def tail(j, _):
            pltpu.sync_copy(ones_v, acc_sh.at[dst_v.at[j]], add=True)
            return 0

        lax.fori_loop((j_n // 8) * 8, j_n, tail, 0)

        plsc.subcore_barrier()
        pltpu.sync_copy(acc_sh.at[pl.ds(s * RPW, RPW)], obuf_v)
        pltpu.sync_copy(obuf_v, out_hbm.at[c, pl.ds(s * RPW, RPW)])

    return deg_kernel


def _make_scatter_kernel(d_feat: int):
    @functools.partial(
        pl.kernel,
        out_type=jax.ShapeDtypeStruct((NC, N_PAD, d_feat), jnp.float32),
        mesh=_mesh,
        compiler_params=_sc_params,
        scratch_types=[
            pltpu.VMEM((J0, CHUNK), jnp.int32),
            pltpu.VMEM((J0, CHUNK), jnp.int32),
            pltpu.VMEM((2, K, CHUNK, d_feat), jnp.float32),
            pltpu.VMEM((RPW, d_feat), jnp.float32),
            pltpu.VMEM_SHARED((N_PAD, d_feat), jnp.float32),
            pltpu.VMEM_SHARED((N_NODES, d_feat), jnp.float32),
            pltpu.SemaphoreType.DMA,
            pltpu.SemaphoreType.DMA,
        ],
    )
    def scat_kernel(g_hbm, src_hbm, dst_hbm, out_hbm,
                    src_v, dst_v, rows_v, buf_v, acc_sh, g_sh, gsem, ssem):
        c = lax.axis_index("c")
        s = lax.axis_index("s")
        g_n = jnp.where(c == 0, G0, G1)
        spl = jnp.where(c == 0, SPL0, SPL1)
        base = jnp.where(c == 0, s * J0, NS * J0 + s * J1)

        pltpu.sync_copy(src_hbm.at[pl.ds(base, J0)], src_v)
        pltpu.sync_copy(dst_hbm.at[pl.ds(base, J0)], dst_v)

        pltpu.sync_copy(g_hbm.at[pl.ds(s * GROWS, GROWS)],
                        g_sh.at[pl.ds(s * GROWS, GROWS)])

        zero16 = jnp.zeros((16,), jnp.float32)

        def zrow(i, _):
            for k in range(d_feat // 16):
                buf_v[i, pl.ds(k * 16, 16)] = zero16
            return 0

        lax.fori_loop(0, RPW, zrow, 0)
        pltpu.sync_copy(buf_v, acc_sh.at[pl.ds(s * RPW, RPW)])
        plsc.subcore_barrier()

        @pl.when(spl > 0)
        def _pro_sh():
            for b in range(K):
                pltpu.async_copy(g_sh.at[src_v.at[b]], rows_v.at[0, b], gsem)

        @pl.when(spl <= 0)
        def _pro_hbm():
            for b in range(K):
                pltpu.async_copy(g_hbm.at[src_v.at[b]], rows_v.at[0, b], gsem)

        def pair(g2, _):
            for p in range(2):
                g = g2 * 2 + p
                for b in range(K):
                    pltpu.make_async_copy(
                        g_hbm.at[src_v.at[0]], rows_v.at[p, b], gsem).wait()
                for b in range(K):
                    pltpu.async_copy(rows_v.at[p, b],
                                     acc_sh.at[dst_v.at[g * K + b]],
                                     ssem, add=True)
                q = 1 - p

                @pl.when(g >= 1)
                def _drain():
                    for b in range(K):
                        pltpu.make_async_copy(
                            g_hbm.at[src_v.at[0]], rows_v.at[q, b],
                            ssem).wait()

                @pl.when((g + 1 < g_n) & (g + 1 < spl))
                def _refill_sh():
                    for b in range(K):
                        pltpu.async_copy(
                            g_sh.at[src_v.at[(g + 1) * K + b]],
                            rows_v.at[q, b], gsem)

                @pl.when((g + 1 < g_n) & (g + 1 >= spl))
                def _refill_hbm():
                    for b in range(K):
                        pltpu.async_copy(
                            g_hbm.at[src_v.at[(g + 1) * K + b]],
                            rows_v.at[q, b], gsem)
            return 0

        lax.fori_loop(0, g_n // 2, pair, 0)
        for b in range(K):
            pltpu.make_async_copy(
                g_hbm.at[src_v.at[0]], rows_v.at[0, b], ssem).wait()

        plsc.subcore_barrier()
        pltpu.sync_copy(acc_sh.at[pl.ds(s * RPW, RPW)], buf_v)
        pltpu.sync_copy(buf_v, out_hbm.at[c, pl.ds(s * RPW, RPW)])

    return scat_kernel


_deg_kernel = _make_deg_kernel()
_scat16 = _make_scatter_kernel(HIDDEN)


def _mm1_body(x_ref, w_ref, h_ref):
    h_ref[...] = jnp.dot(x_ref[...], w_ref[...],
                         preferred_element_type=jnp.float32)


def _dinv_g1_body(degp_ref, h_ref, dinv_ref, g_ref):
    deg = degp_ref[0, :N_NODES] + degp_ref[1, :N_NODES] + 1.0
    dinv = lax.rsqrt(deg).reshape(N_NODES, 1)
    dinv_ref[...] = dinv
    g_ref[...] = h_ref[...] * dinv


def _mid_body(accp_ref, g1_ref, dinv_ref, b1_ref, gz_ref):
    a = (accp_ref[0] + accp_ref[1])[:N_NODES] + g1_ref[...]
    z = jnp.maximum(a * dinv_ref[...] + b1_ref[...], 0.0)
    gz_ref[...] = z * dinv_ref[...]


def _out_body(accp_ref, gz_ref, dinv_ref, b2_ref, w2_ref, out_ref):
    a = (accp_ref[0] + accp_ref[1])[:N_NODES] + gz_ref[...]
    out_ref[...] = (
        jnp.dot(a, w2_ref[...], preferred_element_type=jnp.float32)
        * dinv_ref[...]
        + b2_ref[...]
    )


def kernel(x, edge_index, W1, b1, W2, b2):
    src = edge_index[0]
    dst = edge_index[1]
    pad = E_PAD - N_EDGES
    src2 = jnp.concatenate(
        [src, jnp.zeros((pad,), jnp.int32)]).reshape(NCHUNK, CHUNK)
    dst2 = jnp.concatenate(
        [dst, jnp.full((pad,), DUMMY, jnp.int32)]).reshape(NCHUNK, CHUNK)

    deg_p = _deg_kernel(dst2)
    h1 = pl.pallas_call(
        _mm1_body,
        out_shape=jax.ShapeDtypeStruct((N_NODES, HIDDEN), jnp.float32),
    )(x, W1)

    dinv, g1 = pl.pallas_call(
        _dinv_g1_body,
        out_shape=[
            jax.ShapeDtypeStruct((N_NODES, 1), jnp.float32),
            jax.ShapeDtypeStruct((N_NODES, HIDDEN), jnp.float32),
        ],
    )(deg_p, h1)

    acc1p = _scat16(g1, src2, dst2)

    gz = pl.pallas_call(
        _mid_body,
        out_shape=jax.ShapeDtypeStruct((N_NODES, HIDDEN), jnp.float32),
    )(acc1p, g1, dinv, b1.reshape(1, HIDDEN))

    acczp = _scat16(gz, src2, dst2)

    out = pl.pallas_call(
        _out_body,
        out_shape=jax.ShapeDtypeStruct((N_NODES, D_OUT), jnp.float32),
    )(acczp, gz, dinv, b2.reshape(1, D_OUT), W2)

    return out


# speedup vs baseline: 48.6855x; 1.0618x over previous
import functools

import jax
import jax.numpy as jnp
from jax import lax
from jax.experimental import pallas as pl
from jax.experimental.pallas import tpu as pltpu
from jax.experimental.pallas import tpu_sc as plsc

N_NODES = 10000
N_EDGES = 320000
D_FEAT = 128
HIDDEN = 16
D_OUT = 64

NC = 2
NS = 16
CHUNK = 128
K = 4
NCHUNK = 2696
E_PAD = NCHUNK * CHUNK
N_PAD = 10240
DUMMY = N_NODES
RPW = N_PAD // NS

J0, J1 = 120, 40
G0, G1 = J0 // K, J1 // K
SPL0, SPL1 = 20, 6
GROWS = N_NODES // NS
JD0, JD1 = 124, 39

_mesh = plsc.VectorSubcoreMesh(core_axis_name="c", subcore_axis_name="s")
_sc_params = pltpu.CompilerParams(use_tc_tiling_on_sc=False)


def _make_deg_kernel():
    @functools.partial(
        pl.kernel,
        out_type=jax.ShapeDtypeStruct((NC, N_PAD), jnp.float32),
        mesh=_mesh,
        compiler_params=_sc_params,
        scratch_types=[
            pltpu.VMEM((JD0, CHUNK), jnp.int32),
            pltpu.VMEM((CHUNK,), jnp.float32),
            pltpu.VMEM((RPW,), jnp.float32),
            pltpu.VMEM_SHARED((N_PAD,), jnp.float32),
            pltpu.SemaphoreType.DMA,
        ],
    )
    def deg_kernel(dst_hbm, out_hbm, dst_v, ones_v, obuf_v, acc_sh, sem):
        c = lax.axis_index("c")
        s = lax.axis_index("s")
        j_n = jnp.where(c == 0, JD0, JD1)
        base = jnp.where(c == 0, s * JD0, NS * JD0 + s * JD1)

        pltpu.sync_copy(dst_hbm.at[pl.ds(base, JD0)], dst_v)

        zero16 = jnp.zeros((16,), jnp.float32)
        ones16 = jnp.full((16,), 1.0, jnp.float32)
        for k in range(CHUNK // 16):
            ones_v[pl.ds(k * 16, 16)] = ones16

        def zrow(i, _):
            obuf_v[pl.ds(i * 16, 16)] = zero16
            return 0

        lax.fori_loop(0, RPW // 16, zrow, 0)
        pltpu.sync_copy(obuf_v, acc_sh.at[pl.ds(s * RPW, RPW)])
        plsc.subcore_barrier()

        def group(g, _):
            for b in range(8):
                pltpu.async_copy(ones_v, acc_sh.at[dst_v.at[g * 8 + b]],
                                 sem, add=True)
            for b in range(8):
                pltpu.make_async_copy(ones_v, acc_sh.at[dst_v.at[0]],
                                      sem).wait()
            return 0

        lax.fori_loop(0, j_n // 8, group, 0)

        def tail(j, _):
            pltpu.sync_copy(ones_v, acc_sh.at[dst_v.at[j]], add=True)
            return 0

        lax.fori_loop((j_n // 8) * 8, j_n, tail, 0)

        plsc.subcore_barrier()
        pltpu.sync_copy(acc_sh.at[pl.ds(s * RPW, RPW)], obuf_v)
        pltpu.sync_copy(obuf_v, out_hbm.at[c, pl.ds(s * RPW, RPW)])

    return deg_kernel


def _make_scatter_kernel(d_feat: int):
    @functools.partial(
        pl.kernel,
        out_type=jax.ShapeDtypeStruct((NC, N_PAD, d_feat), jnp.float32),
        mesh=_mesh,
        compiler_params=_sc_params,
        scratch_types=[
            pltpu.VMEM((J0, CHUNK), jnp.int32),
            pltpu.VMEM((J0, CHUNK), jnp.int32),
            pltpu.VMEM((2, K, CHUNK, d_feat), jnp.float32),
            pltpu.VMEM((RPW, d_feat), jnp.float32),
            pltpu.VMEM_SHARED((N_PAD, d_feat), jnp.float32),
            pltpu.VMEM_SHARED((N_NODES, d_feat), jnp.float32),
            pltpu.SemaphoreType.DMA,
            pltpu.SemaphoreType.DMA,
        ],
    )
    def scat_kernel(g_hbm, src_hbm, dst_hbm, out_hbm,
                    src_v, dst_v, rows_v, buf_v, acc_sh, g_sh, gsem, ssem):
        c = lax.axis_index("c")
        s = lax.axis_index("s")
        g_n = jnp.where(c == 0, G0, G1)
        spl = jnp.where(c == 0, SPL0, SPL1)
        base = jnp.where(c == 0, s * J0, NS * J0 + s * J1)

        pltpu.sync_copy(src_hbm.at[pl.ds(base, J0)], src_v)
        pltpu.sync_copy(dst_hbm.at[pl.ds(base, J0)], dst_v)

        pltpu.sync_copy(g_hbm.at[pl.ds(s * GROWS, GROWS)],
                        g_sh.at[pl.ds(s * GROWS, GROWS)])

        zero16 = jnp.zeros((16,), jnp.float32)

        def zrow(i, _):
            for k in range(d_feat // 16):
                buf_v[i, pl.ds(k * 16, 16)] = zero16
            return 0

        lax.fori_loop(0, RPW, zrow, 0)
        pltpu.sync_copy(buf_v, acc_sh.at[pl.ds(s * RPW, RPW)])
        plsc.subcore_barrier()

        @pl.when(spl > 0)
        def _pro_sh():
            for b in range(K):
                pltpu.async_copy(g_sh.at[src_v.at[b]], rows_v.at[0, b], gsem)

        @pl.when(spl <= 0)
        def _pro_hbm():
            for b in range(K):
                pltpu.async_copy(g_hbm.at[src_v.at[b]], rows_v.at[0, b], gsem)

        def pair(g2, _):
            for p in range(2):
                g = g2 * 2 + p
                for b in range(K):
                    pltpu.make_async_copy(
                        g_hbm.at[src_v.at[0]], rows_v.at[p, b], gsem).wait()
                for b in range(K):
                    pltpu.async_copy(rows_v.at[p, b],
                                     acc_sh.at[dst_v.at[g * K + b]],
                                     ssem, add=True)
                q = 1 - p

                @pl.when(g >= 1)
                def _drain():
                    for b in range(K):
                        pltpu.make_async_copy(
                            g_hbm.at[src_v.at[0]], rows_v.at[q, b],
                            ssem).wait()

                @pl.when((g + 1 < g_n) & (g + 1 < spl))
                def _refill_sh():
                    for b in range(K):
                        pltpu.async_copy(
                            g_sh.at[src_v.at[(g + 1) * K + b]],
                            rows_v.at[q, b], gsem)

                @pl.when((g + 1 < g_n) & (g + 1 >= spl))
                def _refill_hbm():
                    for b in range(K):
                        pltpu.async_copy(
                            g_hbm.at[src_v.at[(g + 1) * K + b]],
                            rows_v.at[q, b], gsem)
            return 0

        lax.fori_loop(0, g_n // 2, pair, 0)
        for b in range(K):
            pltpu.make_async_copy(
                g_hbm.at[src_v.at[0]], rows_v.at[0, b], ssem).wait()

        plsc.subcore_barrier()
        pltpu.sync_copy(acc_sh.at[pl.ds(s * RPW, RPW)], buf_v)
        pltpu.sync_copy(buf_v, out_hbm.at[c, pl.ds(s * RPW, RPW)])

    return scat_kernel


_deg_kernel = _make_deg_kernel()
_scat16 = _make_scatter_kernel(HIDDEN)


def _mm1_body(x_ref, w_ref, h_ref):
    h_ref[...] = jnp.dot(x_ref[...], w_ref[...],
                         preferred_element_type=jnp.float32)


def _dinv_g1_body(degp_ref, h_ref, dinv_ref, g_ref):
    deg = degp_ref[0, :N_NODES] + degp_ref[1, :N_NODES] + 1.0
    dinv = lax.rsqrt(deg).reshape(N_NODES, 1)
    dinv_ref[...] = dinv
    g_ref[...] = h_ref[...] * dinv


def _mid_body(accp_ref, g1_ref, dinv_ref, b1_ref, gz_ref):
    a = (accp_ref[0] + accp_ref[1])[:N_NODES] + g1_ref[...]
    z = jnp.maximum(a * dinv_ref[...] + b1_ref[...], 0.0)
    gz_ref[...] = z * dinv_ref[...]


def _out_body(accp_ref, gz_ref, dinv_ref, b2_ref, w2_ref, out_ref):
    a = (accp_ref[0] + accp_ref[1])[:N_NODES] + gz_ref[...]
    out_ref[...] = (
        jnp.dot(a, w2_ref[...], preferred_element_type=jnp.float32)
        * dinv_ref[...]
        + b2_ref[...]
    )


def kernel(x, edge_index, W1, b1, W2, b2):
    src = edge_index[0]
    dst = edge_index[1]
    pad = E_PAD - N_EDGES
    src2 = jnp.concatenate(
        [src, jnp.zeros((pad,), jnp.int32)]).reshape(NCHUNK, CHUNK)
    dst2 = jnp.concatenate(
        [dst, jnp.full((pad,), DUMMY, jnp.int32)]).reshape(NCHUNK, CHUNK)

    deg_p = _deg_kernel(dst2)
    h1 = pl.pallas_call(
        _mm1_body,
        out_shape=jax.ShapeDtypeStruct((N_NODES, HIDDEN), jnp.float32),
    )(x, W1)

    dinv, g1 = pl.pallas_call(
        _dinv_g1_body,
        out_shape=[
            jax.ShapeDtypeStruct((N_NODES, 1), jnp.float32),
            jax.ShapeDtypeStruct((N_NODES, HIDDEN), jnp.float32),
        ],
    )(deg_p, h1)

    acc1p = _scat16(g1, src2, dst2)

    gz = pl.pallas_call(
        _mid_body,
        out_shape=jax.ShapeDtypeStruct((N_NODES, HIDDEN), jnp.float32),
    )(acc1p, g1, dinv, b1.reshape(1, HIDDEN))

    acczp = _scat16(gz, src2, dst2)

    out = pl.pallas_call(
        _out_body,
        out_shape=jax.ShapeDtypeStruct((N_NODES, D_OUT), jnp.float32),
    )(acczp, gz, dinv, b2.reshape(1, D_OUT), W2)

    return out


# speedup vs baseline: 53.3761x; 1.0963x over previous
import functools

import jax
import jax.numpy as jnp
from jax import lax
from jax.experimental import pallas as pl
from jax.experimental.pallas import tpu as pltpu
from jax.experimental.pallas import tpu_sc as plsc

N_NODES = 10000
N_EDGES = 320000
D_FEAT = 128
HIDDEN = 16
D_OUT = 64

NC = 2
NS = 16
CHUNK = 128
K = 4
NCHUNK = 2696
E_PAD = NCHUNK * CHUNK
N_PAD = 10240
DUMMY = N_NODES
RPW = N_PAD // NS

J0, J1 = 120, 40
G0, G1 = J0 // K, J1 // K
SPL0, SPL1 = 24, 8
GROWS = N_NODES // NS
JD0, JD1 = 124, 39

_mesh = plsc.VectorSubcoreMesh(core_axis_name="c", subcore_axis_name="s")
_sc_params = pltpu.CompilerParams(use_tc_tiling_on_sc=False)


def _make_deg_kernel():
    @functools.partial(
        pl.kernel,
        out_type=jax.ShapeDtypeStruct((NC, N_PAD), jnp.float32),
        mesh=_mesh,
        compiler_params=_sc_params,
        scratch_types=[
            pltpu.VMEM((JD0, CHUNK), jnp.int32),
            pltpu.VMEM((CHUNK,), jnp.float32),
            pltpu.VMEM((RPW,), jnp.float32),
            pltpu.VMEM_SHARED((N_PAD,), jnp.float32),
            pltpu.SemaphoreType.DMA,
        ],
    )
    def deg_kernel(dst_hbm, out_hbm, dst_v, ones_v, obuf_v, acc_sh, sem):
        c = lax.axis_index("c")
        s = lax.axis_index("s")
        j_n = jnp.where(c == 0, JD0, JD1)
        base = jnp.where(c == 0, s * JD0, NS * JD0 + s * JD1)

        pltpu.sync_copy(dst_hbm.at[pl.ds(base, JD0)], dst_v)

        zero16 = jnp.zeros((16,), jnp.float32)
        ones16 = jnp.full((16,), 1.0, jnp.float32)
        for k in range(CHUNK // 16):
            ones_v[pl.ds(k * 16, 16)] = ones16

        def zrow(i, _):
            obuf_v[pl.ds(i * 16, 16)] = zero16
            return 0

        lax.fori_loop(0, RPW // 16, zrow, 0)
        pltpu.sync_copy(obuf_v, acc_sh.at[pl.ds(s * RPW, RPW)])
        plsc.subcore_barrier()

        def group(g, _):
            for b in range(8):
                pltpu.async_copy(ones_v, acc_sh.at[dst_v.at[g * 8 + b]],
                                 sem, add=True)
            for b in range(8):
                pltpu.make_async_copy(ones_v, acc_sh.at[dst_v.at[0]],
                                      sem).wait()
            return 0

        lax.fori_loop(0, j_n // 8, group, 0)

        def tail(j, _):
            pltpu.sync_copy(ones_v, acc_sh.at[dst_v.at[j]], add=True)
            return 0

        lax.fori_loop((j_n // 8) * 8, j_n, tail, 0)

        plsc.subcore_barrier()
        pltpu.sync_copy(acc_sh.at[pl.ds(s * RPW, RPW)], obuf_v)
        pltpu.sync_copy(obuf_v, out_hbm.at[c, pl.ds(s * RPW, RPW)])

    return deg_kernel


def _make_scatter_kernel(d_feat: int):
    @functools.partial(
        pl.kernel,
        out_type=jax.ShapeDtypeStruct((NC, N_PAD, d_feat), jnp.float32),
        mesh=_mesh,
        compiler_params=_sc_params,
        scratch_types=[
            pltpu.VMEM((J0, CHUNK), jnp.int32),
            pltpu.VMEM((J0, CHUNK), jnp.int32),
            pltpu.VMEM((2, K, CHUNK, d_feat), jnp.float32),
            pltpu.VMEM((RPW, d_feat), jnp.float32),
            pltpu.VMEM_SHARED((N_PAD, d_feat), jnp.float32),
            pltpu.VMEM_SHARED((N_NODES, d_feat), jnp.float32),
            pltpu.SemaphoreType.DMA,
            pltpu.SemaphoreType.DMA,
        ],
    )
    def scat_kernel(g_hbm, src_hbm, dst_hbm, out_hbm,
                    src_v, dst_v, rows_v, buf_v, acc_sh, g_sh, gsem, ssem):
        c = lax.axis_index("c")
        s = lax.axis_index("s")
        g_n = jnp.where(c == 0, G0, G1)
        spl = jnp.where(c == 0, SPL0, SPL1)
        base = jnp.where(c == 0, s * J0, NS * J0 + s * J1)

        pltpu.sync_copy(src_hbm.at[pl.ds(base, J0)], src_v)
        pltpu.sync_copy(dst_hbm.at[pl.ds(base, J0)], dst_v)

        pltpu.sync_copy(g_hbm.at[pl.ds(s * GROWS, GROWS)],
                        g_sh.at[pl.ds(s * GROWS, GROWS)])

        zero16 = jnp.zeros((16,), jnp.float32)

        def zrow(i, _):
            for k in range(d_feat // 16):
                buf_v[i, pl.ds(k * 16, 16)] = zero16
            return 0

        lax.fori_loop(0, RPW, zrow, 0)
        pltpu.sync_copy(buf_v, acc_sh.at[pl.ds(s * RPW, RPW)])
        plsc.subcore_barrier()

        @pl.when(spl > 0)
        def _pro_sh():
            for b in range(K):
                pltpu.async_copy(g_sh.at[src_v.at[b]], rows_v.at[0, b], gsem)

        @pl.when(spl <= 0)
        def _pro_hbm():
            for b in range(K):
                pltpu.async_copy(g_hbm.at[src_v.at[b]], rows_v.at[0, b], gsem)

        def pair(g2, _):
            for p in range(2):
                g = g2 * 2 + p
                for b in range(K):
                    pltpu.make_async_copy(
                        g_hbm.at[src_v.at[0]], rows_v.at[p, b], gsem).wait()
                for b in range(K):
                    pltpu.async_copy(rows_v.at[p, b],
                                     acc_sh.at[dst_v.at[g * K + b]],
                                     ssem, add=True)
                q = 1 - p

                @pl.when(g >= 1)
                def _drain():
                    for b in range(K):
                        pltpu.make_async_copy(
                            g_hbm.at[src_v.at[0]], rows_v.at[q, b],
                            ssem).wait()

                @pl.when((g + 1 < g_n) & (g + 1 < spl))
                def _refill_sh():
                    for b in range(K):
                        pltpu.async_copy(
                            g_sh.at[src_v.at[(g + 1) * K + b]],
                            rows_v.at[q, b], gsem)

                @pl.when((g + 1 < g_n) & (g + 1 >= spl))
                def _refill_hbm():
                    for b in range(K):
                        pltpu.async_copy(
                            g_hbm.at[src_v.at[(g + 1) * K + b]],
                            rows_v.at[q, b], gsem)
            return 0

        lax.fori_loop(0, g_n // 2, pair, 0)
        for b in range(K):
            pltpu.make_async_copy(
                g_hbm.at[src_v.at[0]], rows_v.at[0, b], ssem).wait()

        plsc.subcore_barrier()
        pltpu.sync_copy(acc_sh.at[pl.ds(s * RPW, RPW)], buf_v)
        pltpu.sync_copy(buf_v, out_hbm.at[c, pl.ds(s * RPW, RPW)])

    return scat_kernel


_deg_kernel = _make_deg_kernel()
_scat16 = _make_scatter_kernel(HIDDEN)


def _mm1_body(x_ref, w_ref, h_ref):
    h_ref[...] = jnp.dot(x_ref[...], w_ref[...],
                         preferred_element_type=jnp.float32)


def _dinv_g1_body(degp_ref, h_ref, dinv_ref, g_ref):
    deg = degp_ref[0, :N_NODES] + degp_ref[1, :N_NODES] + 1.0
    dinv = lax.rsqrt(deg).reshape(N_NODES, 1)
    dinv_ref[...] = dinv
    g_ref[...] = h_ref[...] * dinv


def _mid_body(accp_ref, g1_ref, dinv_ref, b1_ref, gz_ref):
    a = (accp_ref[0] + accp_ref[1])[:N_NODES] + g1_ref[...]
    z = jnp.maximum(a * dinv_ref[...] + b1_ref[...], 0.0)
    gz_ref[...] = z * dinv_ref[...]


def _out_body(accp_ref, gz_ref, dinv_ref, b2_ref, w2_ref, out_ref):
    a = (accp_ref[0] + accp_ref[1])[:N_NODES] + gz_ref[...]
    out_ref[...] = (
        jnp.dot(a, w2_ref[...], preferred_element_type=jnp.float32)
        * dinv_ref[...]
        + b2_ref[...]
    )


def kernel(x, edge_index, W1, b1, W2, b2):
    src = edge_index[0]
    dst = edge_index[1]
    pad = E_PAD - N_EDGES
    src2 = jnp.concatenate(
        [src, jnp.zeros((pad,), jnp.int32)]).reshape(NCHUNK, CHUNK)
    dst2 = jnp.concatenate(
        [dst, jnp.full((pad,), DUMMY, jnp.int32)]).reshape(NCHUNK, CHUNK)

    deg_p = _deg_kernel(dst2)
    h1 = pl.pallas_call(
        _mm1_body,
        out_shape=jax.ShapeDtypeStruct((N_NODES, HIDDEN), jnp.float32),
    )(x, W1)

    dinv, g1 = pl.pallas_call(
        _dinv_g1_body,
        out_shape=[
            jax.ShapeDtypeStruct((N_NODES, 1), jnp.float32),
            jax.ShapeDtypeStruct((N_NODES, HIDDEN), jnp.float32),
        ],
    )(deg_p, h1)

    acc1p = _scat16(g1, src2, dst2)

    gz = pl.pallas_call(
        _mid_body,
        out_shape=jax.ShapeDtypeStruct((N_NODES, HIDDEN), jnp.float32),
    )(acc1p, g1, dinv, b1.reshape(1, HIDDEN))

    acczp = _scat16(gz, src2, dst2)

    out = pl.pallas_call(
        _out_body,
        out_shape=jax.ShapeDtypeStruct((N_NODES, D_OUT), jnp.float32),
    )(acczp, gz, dinv, b2.reshape(1, D_OUT), W2)

    return out


# speedup vs baseline: 55.1413x; 1.0331x over previous
import functools

import jax
import jax.numpy as jnp
from jax import lax
from jax.experimental import pallas as pl
from jax.experimental.pallas import tpu as pltpu
from jax.experimental.pallas import tpu_sc as plsc

N_NODES = 10000
N_EDGES = 320000
D_FEAT = 128
HIDDEN = 16
D_OUT = 64

NC = 2
NS = 16
CHUNK = 128
K = 4
NCHUNK = 2696
E_PAD = NCHUNK * CHUNK
N_PAD = 10240
DUMMY = N_NODES
RPW = N_PAD // NS

J0, J1 = 120, 40
G0, G1 = J0 // K, J1 // K
SPL0, SPL1 = 30, 10
GROWS = N_NODES // NS
JD0, JD1 = 124, 39

_mesh = plsc.VectorSubcoreMesh(core_axis_name="c", subcore_axis_name="s")
_sc_params = pltpu.CompilerParams(use_tc_tiling_on_sc=False)


def _make_deg_kernel():
    @functools.partial(
        pl.kernel,
        out_type=jax.ShapeDtypeStruct((NC, N_PAD), jnp.float32),
        mesh=_mesh,
        compiler_params=_sc_params,
        scratch_types=[
            pltpu.VMEM((JD0, CHUNK), jnp.int32),
            pltpu.VMEM((CHUNK,), jnp.float32),
            pltpu.VMEM((RPW,), jnp.float32),
            pltpu.VMEM_SHARED((N_PAD,), jnp.float32),
            pltpu.SemaphoreType.DMA,
        ],
    )
    def deg_kernel(dst_hbm, out_hbm, dst_v, ones_v, obuf_v, acc_sh, sem):
        c = lax.axis_index("c")
        s = lax.axis_index("s")
        j_n = jnp.where(c == 0, JD0, JD1)
        base = jnp.where(c == 0, s * JD0, NS * JD0 + s * JD1)

        pltpu.sync_copy(dst_hbm.at[pl.ds(base, JD0)], dst_v)

        zero16 = jnp.zeros((16,), jnp.float32)
        ones16 = jnp.full((16,), 1.0, jnp.float32)
        for k in range(CHUNK // 16):
            ones_v[pl.ds(k * 16, 16)] = ones16

        def zrow(i, _):
            obuf_v[pl.ds(i * 16, 16)] = zero16
            return 0

        lax.fori_loop(0, RPW // 16, zrow, 0)
        pltpu.sync_copy(obuf_v, acc_sh.at[pl.ds(s * RPW, RPW)])
        plsc.subcore_barrier()

        def group(g, _):
            for b in range(8):
                pltpu.async_copy(ones_v, acc_sh.at[dst_v.at[g * 8 + b]],
                                 sem, add=True)
            for b in range(8):
                pltpu.make_async_copy(ones_v, acc_sh.at[dst_v.at[0]],
                                      sem).wait()
            return 0

        lax.fori_loop(0, j_n // 8, group, 0)

        def tail(j, _):
            pltpu.sync_copy(ones_v, acc_sh.at[dst_v.at[j]], add=True)
            return 0

        lax.fori_loop((j_n // 8) * 8, j_n, tail, 0)

        plsc.subcore_barrier()
        pltpu.sync_copy(acc_sh.at[pl.ds(s * RPW, RPW)], obuf_v)
        pltpu.sync_copy(obuf_v, out_hbm.at[c, pl.ds(s * RPW, RPW)])

    return deg_kernel


def _make_scatter_kernel(d_feat: int):
    @functools.partial(
        pl.kernel,
        out_type=jax.ShapeDtypeStruct((NC, N_PAD, d_feat), jnp.float32),
        mesh=_mesh,
        compiler_params=_sc_params,
        scratch_types=[
            pltpu.VMEM((J0, CHUNK), jnp.int32),
            pltpu.VMEM((J0, CHUNK), jnp.int32),
            pltpu.VMEM((2, K, CHUNK, d_feat), jnp.float32),
            pltpu.VMEM((RPW, d_feat), jnp.float32),
            pltpu.VMEM_SHARED((N_PAD, d_feat), jnp.float32),
            pltpu.VMEM_SHARED((N_NODES, d_feat), jnp.float32),
            pltpu.SemaphoreType.DMA,
            pltpu.SemaphoreType.DMA,
        ],
    )
    def scat_kernel(g_hbm, src_hbm, dst_hbm, out_hbm,
                    src_v, dst_v, rows_v, buf_v, acc_sh, g_sh, gsem, ssem):
        c = lax.axis_index("c")
        s = lax.axis_index("s")
        g_n = jnp.where(c == 0, G0, G1)
        spl = jnp.where(c == 0, SPL0, SPL1)
        base = jnp.where(c == 0, s * J0, NS * J0 + s * J1)

        pltpu.sync_copy(src_hbm.at[pl.ds(base, J0)], src_v)
        pltpu.sync_copy(dst_hbm.at[pl.ds(base, J0)], dst_v)

        pltpu.sync_copy(g_hbm.at[pl.ds(s * GROWS, GROWS)],
                        g_sh.at[pl.ds(s * GROWS, GROWS)])

        zero16 = jnp.zeros((16,), jnp.float32)

        def zrow(i, _):
            for k in range(d_feat // 16):
                buf_v[i, pl.ds(k * 16, 16)] = zero16
            return 0

        lax.fori_loop(0, RPW, zrow, 0)
        pltpu.sync_copy(buf_v, acc_sh.at[pl.ds(s * RPW, RPW)])
        plsc.subcore_barrier()

        @pl.when(spl > 0)
        def _pro_sh():
            for b in range(K):
                pltpu.async_copy(g_sh.at[src_v.at[b]], rows_v.at[0, b], gsem)

        @pl.when(spl <= 0)
        def _pro_hbm():
            for b in range(K):
                pltpu.async_copy(g_hbm.at[src_v.at[b]], rows_v.at[0, b], gsem)

        def pair(g2, _):
            for p in range(2):
                g = g2 * 2 + p
                for b in range(K):
                    pltpu.make_async_copy(
                        g_hbm.at[src_v.at[0]], rows_v.at[p, b], gsem).wait()
                for b in range(K):
                    pltpu.async_copy(rows_v.at[p, b],
                                     acc_sh.at[dst_v.at[g * K + b]],
                                     ssem, add=True)
                q = 1 - p

                @pl.when(g >= 1)
                def _drain():
                    for b in range(K):
                        pltpu.make_async_copy(
                            g_hbm.at[src_v.at[0]], rows_v.at[q, b],
                            ssem).wait()

                @pl.when((g + 1 < g_n) & (g + 1 < spl))
                def _refill_sh():
                    for b in range(K):
                        pltpu.async_copy(
                            g_sh.at[src_v.at[(g + 1) * K + b]],
                            rows_v.at[q, b], gsem)

                @pl.when((g + 1 < g_n) & (g + 1 >= spl))
                def _refill_hbm():
                    for b in range(K):
                        pltpu.async_copy(
                            g_hbm.at[src_v.at[(g + 1) * K + b]],
                            rows_v.at[q, b], gsem)
            return 0

        lax.fori_loop(0, g_n // 2, pair, 0)
        for b in range(K):
            pltpu.make_async_copy(
                g_hbm.at[src_v.at[0]], rows_v.at[0, b], ssem).wait()

        plsc.subcore_barrier()
        pltpu.sync_copy(acc_sh.at[pl.ds(s * RPW, RPW)], buf_v)
        pltpu.sync_copy(buf_v, out_hbm.at[c, pl.ds(s * RPW, RPW)])

    return scat_kernel


_deg_kernel = _make_deg_kernel()
_scat16 = _make_scatter_kernel(HIDDEN)


def _mm1_body(x_ref, w_ref, h_ref):
    h_ref[...] = jnp.dot(x_ref[...], w_ref[...],
                         preferred_element_type=jnp.float32)


def _dinv_g1_body(degp_ref, h_ref, dinv_ref, g_ref):
    deg = degp_ref[0, :N_NODES] + degp_ref[1, :N_NODES] + 1.0
    dinv = lax.rsqrt(deg).reshape(N_NODES, 1)
    dinv_ref[...] = dinv
    g_ref[...] = h_ref[...] * dinv


def _mid_body(accp_ref, g1_ref, dinv_ref, b1_ref, gz_ref):
    a = (accp_ref[0] + accp_ref[1])[:N_NODES] + g1_ref[...]
    z = jnp.maximum(a * dinv_ref[...] + b1_ref[...], 0.0)
    gz_ref[...] = z * dinv_ref[...]


def _out_body(accp_ref, gz_ref, dinv_ref, b2_ref, w2_ref, out_ref):
    a = (accp_ref[0] + accp_ref[1])[:N_NODES] + gz_ref[...]
    out_ref[...] = (
        jnp.dot(a, w2_ref[...], preferred_element_type=jnp.float32)
        * dinv_ref[...]
        + b2_ref[...]
    )


def kernel(x, edge_index, W1, b1, W2, b2):
    src = edge_index[0]
    dst = edge_index[1]
    pad = E_PAD - N_EDGES
    src2 = jnp.concatenate(
        [src, jnp.zeros((pad,), jnp.int32)]).reshape(NCHUNK, CHUNK)
    dst2 = jnp.concatenate(
        [dst, jnp.full((pad,), DUMMY, jnp.int32)]).reshape(NCHUNK, CHUNK)

    deg_p = _deg_kernel(dst2)
    h1 = pl.pallas_call(
        _mm1_body,
        out_shape=jax.ShapeDtypeStruct((N_NODES, HIDDEN), jnp.float32),
    )(x, W1)

    dinv, g1 = pl.pallas_call(
        _dinv_g1_body,
        out_shape=[
            jax.ShapeDtypeStruct((N_NODES, 1), jnp.float32),
            jax.ShapeDtypeStruct((N_NODES, HIDDEN), jnp.float32),
        ],
    )(deg_p, h1)

    acc1p = _scat16(g1, src2, dst2)

    gz = pl.pallas_call(
        _mid_body,
        out_shape=jax.ShapeDtypeStruct((N_NODES, HIDDEN), jnp.float32),
    )(acc1p, g1, dinv, b1.reshape(1, HIDDEN))

    acczp = _scat16(gz, src2, dst2)

    out = pl.pallas_call(
        _out_body,
        out_shape=jax.ShapeDtypeStruct((N_NODES, D_OUT), jnp.float32),
    )(acczp, gz, dinv, b2.reshape(1, D_OUT), W2)

    return out


# speedup vs baseline: 57.6150x; 1.0449x over previous
import functools

import jax
import jax.numpy as jnp
from jax import lax
from jax.experimental import pallas as pl
from jax.experimental.pallas import tpu as pltpu
from jax.experimental.pallas import tpu_sc as plsc

N_NODES = 10000
N_EDGES = 320000
D_FEAT = 128
HIDDEN = 16
D_OUT = 64

NC = 2
NS = 16
CHUNK = 128
K = 4
NCHUNK = 2616
E_PAD = NCHUNK * CHUNK
N_PAD = 10240
DUMMY = N_NODES
RPW = N_PAD // NS

J0, J1 = 80, 80
G0, G1 = J0 // K, J1 // K
GROWS = N_NODES // NS
JD0, JD1 = 82, 81

_mesh = plsc.VectorSubcoreMesh(core_axis_name="c", subcore_axis_name="s")
_sc_params = pltpu.CompilerParams(use_tc_tiling_on_sc=False)


def _make_deg_kernel():
    @functools.partial(
        pl.kernel,
        out_type=jax.ShapeDtypeStruct((NC, N_PAD), jnp.float32),
        mesh=_mesh,
        compiler_params=_sc_params,
        scratch_types=[
            pltpu.VMEM((JD0, CHUNK), jnp.int32),
            pltpu.VMEM((CHUNK,), jnp.float32),
            pltpu.VMEM((RPW,), jnp.float32),
            pltpu.VMEM_SHARED((N_PAD,), jnp.float32),
            pltpu.SemaphoreType.DMA,
        ],
    )
    def deg_kernel(dst_hbm, out_hbm, dst_v, ones_v, obuf_v, acc_sh, sem):
        c = lax.axis_index("c")
        s = lax.axis_index("s")
        j_n = jnp.where(c == 0, JD0, JD1)
        base = jnp.where(c == 0, s * JD0, NS * JD0 + s * JD1)

        pltpu.sync_copy(dst_hbm.at[pl.ds(base, JD0)], dst_v)

        zero16 = jnp.zeros((16,), jnp.float32)
        ones16 = jnp.full((16,), 1.0, jnp.float32)
        for k in range(CHUNK // 16):
            ones_v[pl.ds(k * 16, 16)] = ones16

        def zrow(i, _):
            obuf_v[pl.ds(i * 16, 16)] = zero16
            return 0

        lax.fori_loop(0, RPW // 16, zrow, 0)
        pltpu.sync_copy(obuf_v, acc_sh.at[pl.ds(s * RPW, RPW)])
        plsc.subcore_barrier()

        def group(g, _):
            for b in range(8):
                pltpu.async_copy(ones_v, acc_sh.at[dst_v.at[g * 8 + b]],
                                 sem, add=True)
            for b in range(8):
                pltpu.make_async_copy(ones_v, acc_sh.at[dst_v.at[0]],
                                      sem).wait()
            return 0

        lax.fori_loop(0, j_n // 8, group, 0)

        def tail(j, _):
            pltpu.sync_copy(ones_v, acc_sh.at[dst_v.at[j]], add=True)
            return 0

        lax.fori_loop((j_n // 8) * 8, j_n, tail, 0)

        plsc.subcore_barrier()
        pltpu.sync_copy(acc_sh.at[pl.ds(s * RPW, RPW)], obuf_v)
        pltpu.sync_copy(obuf_v, out_hbm.at[c, pl.ds(s * RPW, RPW)])

    return deg_kernel


def _make_scatter_kernel(d_feat: int):
    @functools.partial(
        pl.kernel,
        out_type=jax.ShapeDtypeStruct((NC, N_PAD, d_feat), jnp.float32),
        mesh=_mesh,
        compiler_params=_sc_params,
        scratch_types=[
            pltpu.VMEM((J0, CHUNK), jnp.int32),
            pltpu.VMEM((J0, CHUNK), jnp.int32),
            pltpu.VMEM((2, K, CHUNK, d_feat), jnp.float32),
            pltpu.VMEM((RPW, d_feat), jnp.float32),
            pltpu.VMEM_SHARED((N_PAD, d_feat), jnp.float32),
            pltpu.VMEM_SHARED((N_NODES, d_feat), jnp.float32),
            pltpu.SemaphoreType.DMA,
            pltpu.SemaphoreType.DMA,
        ],
    )
    def scat_kernel(g_hbm, src_hbm, dst_hbm, out_hbm,
                    src_v, dst_v, rows_v, buf_v, acc_sh, g_sh, gsem, ssem):
        c = lax.axis_index("c")
        s = lax.axis_index("s")
        g_n = jnp.where(c == 0, G0, G1)
        base = jnp.where(c == 0, s * J0, NS * J0 + s * J1)

        pltpu.sync_copy(src_hbm.at[pl.ds(base, J0)], src_v)
        pltpu.sync_copy(dst_hbm.at[pl.ds(base, J0)], dst_v)

        pltpu.sync_copy(g_hbm.at[pl.ds(s * GROWS, GROWS)],
                        g_sh.at[pl.ds(s * GROWS, GROWS)])

        zero16 = jnp.zeros((16,), jnp.float32)

        def zrow(i, _):
            for k in range(d_feat // 16):
                buf_v[i, pl.ds(k * 16, 16)] = zero16
            return 0

        lax.fori_loop(0, RPW, zrow, 0)
        pltpu.sync_copy(buf_v, acc_sh.at[pl.ds(s * RPW, RPW)])
        plsc.subcore_barrier()

        for b in range(K):
            pltpu.async_copy(g_sh.at[src_v.at[b]], rows_v.at[0, b], gsem)

        def pair(g2, _):
            for p in range(2):
                g = g2 * 2 + p
                for b in range(K):
                    pltpu.make_async_copy(
                        g_hbm.at[src_v.at[0]], rows_v.at[p, b], gsem).wait()
                for b in range(K):
                    pltpu.async_copy(rows_v.at[p, b],
                                     acc_sh.at[dst_v.at[g * K + b]],
                                     ssem, add=True)
                q = 1 - p

                @pl.when(g >= 1)
                def _drain():
                    for b in range(K):
                        pltpu.make_async_copy(
                            g_hbm.at[src_v.at[0]], rows_v.at[q, b],
                            ssem).wait()

                @pl.when(g + 1 < g_n)
                def _refill():
                    for b in range(K):
                        pltpu.async_copy(
                            g_sh.at[src_v.at[(g + 1) * K + b]],
                            rows_v.at[q, b], gsem)
            return 0

        lax.fori_loop(0, g_n // 2, pair, 0)
        for b in range(K):
            pltpu.make_async_copy(
                g_hbm.at[src_v.at[0]], rows_v.at[0, b], ssem).wait()

        plsc.subcore_barrier()
        pltpu.sync_copy(acc_sh.at[pl.ds(s * RPW, RPW)], buf_v)
        pltpu.sync_copy(buf_v, out_hbm.at[c, pl.ds(s * RPW, RPW)])

    return scat_kernel


_deg_kernel = _make_deg_kernel()
_scat16 = _make_scatter_kernel(HIDDEN)


def _mm1_body(x_ref, w_ref, h_ref):
    h_ref[...] = jnp.dot(x_ref[...], w_ref[...],
                         preferred_element_type=jnp.float32)


def _dinv_g1_body(degp_ref, h_ref, dinv_ref, g_ref):
    deg = degp_ref[0, :N_NODES] + degp_ref[1, :N_NODES] + 1.0
    dinv = lax.rsqrt(deg).reshape(N_NODES, 1)
    dinv_ref[...] = dinv
    g_ref[...] = h_ref[...] * dinv


def _mid_body(accp_ref, g1_ref, dinv_ref, b1_ref, gz_ref):
    a = (accp_ref[0] + accp_ref[1])[:N_NODES] + g1_ref[...]
    z = jnp.maximum(a * dinv_ref[...] + b1_ref[...], 0.0)
    gz_ref[...] = z * dinv_ref[...]


def _out_body(accp_ref, gz_ref, dinv_ref, b2_ref, w2_ref, out_ref):
    a = (accp_ref[0] + accp_ref[1])[:N_NODES] + gz_ref[...]
    out_ref[...] = (
        jnp.dot(a, w2_ref[...], preferred_element_type=jnp.float32)
        * dinv_ref[...]
        + b2_ref[...]
    )


def kernel(x, edge_index, W1, b1, W2, b2):
    src = edge_index[0]
    dst = edge_index[1]
    pad = E_PAD - N_EDGES
    src2 = jnp.concatenate(
        [src, jnp.zeros((pad,), jnp.int32)]).reshape(NCHUNK, CHUNK)
    dst2 = jnp.concatenate(
        [dst, jnp.full((pad,), DUMMY, jnp.int32)]).reshape(NCHUNK, CHUNK)

    deg_p = _deg_kernel(dst2)
    h1 = pl.pallas_call(
        _mm1_body,
        out_shape=jax.ShapeDtypeStruct((N_NODES, HIDDEN), jnp.float32),
    )(x, W1)

    dinv, g1 = pl.pallas_call(
        _dinv_g1_body,
        out_shape=[
            jax.ShapeDtypeStruct((N_NODES, 1), jnp.float32),
            jax.ShapeDtypeStruct((N_NODES, HIDDEN), jnp.float32),
        ],
    )(deg_p, h1)

    acc1p = _scat16(g1, src2, dst2)

    gz = pl.pallas_call(
        _mid_body,
        out_shape=jax.ShapeDtypeStruct((N_NODES, HIDDEN), jnp.float32),
    )(acc1p, g1, dinv, b1.reshape(1, HIDDEN))

    acczp = _scat16(gz, src2, dst2)

    out = pl.pallas_call(
        _out_body,
        out_shape=jax.ShapeDtypeStruct((N_NODES, D_OUT), jnp.float32),
    )(acczp, gz, dinv, b2.reshape(1, D_OUT), W2)

    return out
